# baseline (speedup 1.0000x reference)
import dataclasses
import functools

import jax
import jax.numpy as jnp
from jax import lax
from jax.experimental import pallas as pl
from jax.experimental.pallas import tpu as pltpu
from jax.experimental.pallas import tpu_sc as plsc

_NC = 2
_NS = 16
_K = 80
_G = 64
_ZR = 16


def _dotT(a, b):
  return lax.dot_general(a, b, (((1,), (1,)), ((), ())),
                         preferred_element_type=jnp.float32,
                         precision=lax.Precision.HIGHEST)


def _sc_segsum(u, src, dst, with_cnt):
  N, D = u.shape
  E = src.shape[0]
  epw = E // (_NC * _NS)
  nchunks = epw // _K
  rps = (N // _NS) // 8 * 8
  rem = N - rps * _NS
  lanes = D // 16
  nsub = _K // 16
  CB = N
  AGG = (N + N // 8 + 16 * _NS * 2 - 1) // (16 * _NS) * (16 * _NS)
  zrows = AGG // _NS
  NCNT = 640 * _NS

  mesh = plsc.VectorSubcoreMesh(core_axis_name="c", subcore_axis_name="s")

  out_type = [jax.ShapeDtypeStruct((_NC, N, D), jnp.float32)]
  scratch = [
      pltpu.VMEM_SHARED((AGG if with_cnt else N, D), jnp.float32),
      pltpu.VMEM((_K,), jnp.int32),
      pltpu.VMEM((_K,), jnp.int32),
      pltpu.VMEM((_K,), jnp.int32),
      pltpu.VMEM((_K,), jnp.int32),
      pltpu.VMEM((_K, D), jnp.float32),
      pltpu.VMEM((_K, D), jnp.float32),
      pltpu.VMEM((_ZR, D), jnp.float32),
      pltpu.SemaphoreType.DMA,
      pltpu.SemaphoreType.DMA,
      pltpu.SemaphoreType.DMA,
      pltpu.SemaphoreType.DMA,
  ]
  if with_cnt:
    out_type.append(jax.ShapeDtypeStruct((_NC, NCNT, D), jnp.float32))
    scratch += [
        pltpu.VMEM((_K,), jnp.int32),
        pltpu.VMEM((_K,), jnp.int32),
        pltpu.VMEM((_K, D), jnp.float32),
        pltpu.VMEM((16, D), jnp.float32),
        pltpu.VMEM((16, D), jnp.float32),
        pltpu.SemaphoreType.DMA,
    ]

  cp = pltpu.CompilerParams()
  if "needs_layout_passes" in pltpu.CompilerParams.__dataclass_fields__:
    cp = dataclasses.replace(cp, needs_layout_passes=False)

  @functools.partial(pl.kernel, out_type=out_type, mesh=mesh,
                     scratch_types=scratch, compiler_params=cp)
  def k(*refs):
    if with_cnt:
      (u_hbm, src_hbm, dst_hbm, out_hbm, cnt_hbm,
       agg_sh, srcv0, dstv0, srcv1, dstv1, rows0, rows1, zbuf,
       si0, si1, sg0, sg1, cidxv, lane0v, cntrows, cbuf, ebuf, scnt) = refs
    else:
      (u_hbm, src_hbm, dst_hbm, out_hbm,
       agg_sh, srcv0, dstv0, srcv1, dstv1, rows0, rows1, zbuf,
       si0, si1, sg0, sg1) = refs

    c = lax.axis_index("c")
    s = lax.axis_index("s")
    ebase = (c * _NS + s) * epw
    rbase = s * rps

    zv = jnp.zeros((16,), jnp.float32)
    ov = jnp.ones((16,), jnp.float32)

    @pl.loop(0, _ZR)
    def _(r):
      @pl.loop(0, lanes)
      def _(j):
        zbuf[r, pl.ds(j * 16, 16)] = zv

    nzt = (zrows if with_cnt else rps + rem) // _ZR
    zb = s * zrows if with_cnt else rbase

    @pl.loop(0, nzt)
    def _(t):
      pltpu.async_copy(zbuf, agg_sh.at[pl.ds(zb + t * _ZR, _ZR)], sg0)

    @pl.loop(0, nzt)
    def _(t):
      pltpu.make_async_copy(zbuf, agg_sh.at[pl.ds(zb + t * _ZR, _ZR)],
                            sg0).wait()

    if with_cnt:
      @pl.loop(0, _K)
      def _(r):
        @pl.loop(0, lanes)
        def _(j):
          cntrows[r, pl.ds(j * 16, 16)] = zv

      cbv = jnp.full((16,), CB, jnp.int32)
      z16 = jnp.zeros((16,), jnp.int32)
      for j in range(nsub):
        cidxv[pl.ds(j * 16, 16)] = cbv
        lane0v[pl.ds(j * 16, 16)] = z16
      pltpu.async_copy(cntrows, agg_sh.at[cidxv], scnt, add=True)

    plsc.subcore_barrier()

    def scatter_chunk(rows_b, dstv_b):
      pltpu.sync_copy(rows_b, agg_sh.at[dstv_b], add=True)
      if with_cnt:
        pltpu.make_async_copy(cntrows, agg_sh.at[cidxv], scnt).wait()

        @pl.loop(0, nsub)
        def _(j):
          rowi = lax.broadcasted_iota(jnp.int32, (16,), 0) + j * 16
          l0 = lane0v[pl.ds(j * 16, 16)]
          for kk in range(16):
            plsc.store_scatter(cntrows, [rowi, l0 + kk], zv)
          d = dstv_b[pl.ds(j * 16, 16)]
          cidxv[pl.ds(j * 16, 16)] = lax.shift_right_logical(d, 3) + CB
          lane0 = (d & 7) * 16
          lane0v[pl.ds(j * 16, 16)] = lane0
          for kk in range(16):
            plsc.store_scatter(cntrows, [rowi, lane0 + kk], ov)

        pltpu.async_copy(cntrows, agg_sh.at[cidxv], scnt, add=True)

    def start_idx(base, sv, dv, sem):
      pltpu.async_copy(src_hbm.at[pl.ds(base, _K)], sv, sem)
      pltpu.async_copy(dst_hbm.at[pl.ds(base, _K)], dv, sem)

    def wait_idx(sv, dv, sem):
      pltpu.make_async_copy(src_hbm.at[pl.ds(0, _K)], sv, sem).wait()
      pltpu.make_async_copy(dst_hbm.at[pl.ds(0, _K)], dv, sem).wait()

    def wait_gather(rows_b, sem):
      pltpu.make_async_copy(u_hbm.at[pl.ds(0, _K)], rows_b, sem).wait()

    assert nchunks % 2 == 1
    npairs = nchunks // 2

    start_idx(ebase, srcv0, dstv0, si0)
    start_idx(ebase + _K, srcv1, dstv1, si1)
    wait_idx(srcv0, dstv0, si0)
    pltpu.async_copy(u_hbm.at[srcv0], rows0, sg0)

    @pl.loop(0, npairs)
    def _(t):
      base_c = ebase + (2 * t + 2) * _K
      base_d = ebase + (2 * t + 3) * _K
      wait_gather(rows0, sg0)
      wait_idx(srcv1, dstv1, si1)
      pltpu.async_copy(u_hbm.at[srcv1], rows1, sg1)
      scatter_chunk(rows0, dstv0)
      start_idx(base_c, srcv0, dstv0, si0)
      wait_gather(rows1, sg1)
      wait_idx(srcv0, dstv0, si0)
      pltpu.async_copy(u_hbm.at[srcv0], rows0, sg0)
      scatter_chunk(rows1, dstv1)

      @pl.when(t < npairs - 1)
      def _():
        start_idx(base_d, srcv1, dstv1, si1)

    wait_gather(rows0, sg0)
    scatter_chunk(rows0, dstv0)
    if with_cnt:
      pltpu.make_async_copy(cntrows, agg_sh.at[cidxv], scnt).wait()

    plsc.subcore_barrier()

    pltpu.sync_copy(agg_sh.at[pl.ds(rbase, rps + rem)],
                    out_hbm.at[c].at[pl.ds(rbase, rps + rem)])

    if with_cnt:
      nbase = s * 640

      @pl.loop(0, 5)
      def _(t):
        pltpu.sync_copy(agg_sh.at[pl.ds(CB + s * 80 + t * 16, 16)], cbuf)
        for g in range(8):
          for r in range(16):
            cv = cbuf[2 * g + r // 8, pl.ds((r % 8) * 16, 16)]
            for j in range(lanes):
              ebuf[r, pl.ds(j * 16, 16)] = cv
          pltpu.sync_copy(
              ebuf, cnt_hbm.at[c].at[pl.ds(nbase + t * 128 + g * 16, 16)])

  res = k(u, src, dst)
  return (res[0], res[1]) if with_cnt else res[0]


def _finish_layer(p_ref, cnt_ref, r, wl_ref):
  p = p_ref[...]
  s = p[0] + p[1]
  cnt = cnt_ref[...]
  c2 = cnt[0] + cnt[1]
  m = s / jnp.maximum(c2, 1.0)
  return jnp.maximum(_dotT(m, wl_ref[...]) + r, 0.0)


def _tc_b(p, cnt, x, Wl1, Wr1, bl1, Wr2, bl2, R):
  _, N, H = p.shape
  nb = N // R

  def body(p_ref, cnt_ref, x_ref, wl_ref, wr1_ref, bl1_ref, wr_ref, bl_ref,
           h_ref, r_ref):
    r1 = _dotT(x_ref[...], wr1_ref[...]) + bl1_ref[...]
    h = _finish_layer(p_ref, cnt_ref, r1, wl_ref)
    h_ref[...] = h
    r_ref[...] = _dotT(h, wr_ref[...]) + bl_ref[...]

  return pl.pallas_call(
      body,
      grid=(nb,),
      in_specs=[
          pl.BlockSpec((2, R, H), lambda i: (0, i, 0)),
          pl.BlockSpec((2, R, H), lambda i: (0, i, 0)),
          pl.BlockSpec((R, H), lambda i: (i, 0)),
          pl.BlockSpec((H, H), lambda i: (0, 0)),
          pl.BlockSpec((H, H), lambda i: (0, 0)),
          pl.BlockSpec((1, H), lambda i: (0, 0)),
          pl.BlockSpec((H, H), lambda i: (0, 0)),
          pl.BlockSpec((1, H), lambda i: (0, 0)),
      ],
      out_specs=[
          pl.BlockSpec((R, H), lambda i: (i, 0)),
          pl.BlockSpec((R, H), lambda i: (i, 0)),
      ],
      out_shape=[
          jax.ShapeDtypeStruct((N, H), jnp.float32),
          jax.ShapeDtypeStruct((N, H), jnp.float32),
      ],
  )(p, cnt, x, Wl1, Wr1, bl1.reshape(1, H), Wr2, bl2.reshape(1, H))


def _tc_c(p, cnt, r2, Wl2, batch3, Wfc, bfc, R):
  _, N, H = p.shape
  C = Wfc.shape[0]
  nb = N // R

  def body(p_ref, cnt_ref, r2_ref, wl_ref, b_ref, wfc_ref, bfc_ref, out_ref,
           acc, cntg):
    i = pl.program_id(0)

    @pl.when(i == 0)
    def _():
      acc[...] = jnp.zeros_like(acc)
      cntg[...] = jnp.zeros_like(cntg)

    h = _finish_layer(p_ref, cnt_ref, r2_ref[...], wl_ref)
    b = b_ref[0]
    gid = lax.broadcasted_iota(jnp.int32, (_G, R), 0)
    oh = (jnp.broadcast_to(b, (_G, R)) == gid).astype(jnp.float32)
    acc[...] += lax.dot_general(oh, h, (((1,), (0,)), ((), ())),
                                preferred_element_type=jnp.float32,
                                precision=lax.Precision.HIGHEST)
    cntg[...] += jnp.broadcast_to(jnp.sum(oh, axis=1, keepdims=True), (_G, H))

    @pl.when(i == nb - 1)
    def _():
      pooled = acc[...] / jnp.maximum(cntg[...], 1.0)
      out_ref[...] = _dotT(pooled, wfc_ref[...]) + bfc_ref[...]

  return pl.pallas_call(
      body,
      grid=(nb,),
      in_specs=[
          pl.BlockSpec((2, R, H), lambda i: (0, i, 0)),
          pl.BlockSpec((2, R, H), lambda i: (0, i, 0)),
          pl.BlockSpec((R, H), lambda i: (i, 0)),
          pl.BlockSpec((H, H), lambda i: (0, 0)),
          pl.BlockSpec((1, 1, R), lambda i: (i, 0, 0)),
          pl.BlockSpec((C, H), lambda i: (0, 0)),
          pl.BlockSpec((1, C), lambda i: (0, 0)),
      ],
      out_specs=pl.BlockSpec((_G, C), lambda i: (0, 0)),
      out_shape=jax.ShapeDtypeStruct((_G, C), jnp.float32),
      scratch_shapes=[
          pltpu.VMEM((_G, H), jnp.float32),
          pltpu.VMEM((_G, H), jnp.float32),
      ],
  )(p, cnt, r2, Wl2, batch3, Wfc, bfc.reshape(1, C))


def kernel(x, edge_index, batch, Wl1, bl1, Wr1, Wl2, bl2, Wr2, Wfc, bfc):
  N, D = x.shape
  R = 1000

  src = edge_index[0]
  dst = edge_index[1]
  batch3 = batch.reshape(N // R, 1, R)

  s1, cnt = _sc_segsum(x, src, dst, with_cnt=True)
  h1, r2 = _tc_b(s1, cnt, x, Wl1, Wr1, bl1, Wr2, bl2, R)
  s2 = _sc_segsum(h1, src, dst, with_cnt=False)
  return _tc_c(s2, cnt, r2, Wl2, batch3, Wfc, bfc, R)


# speedup vs baseline: 1.0193x; 1.0193x over previous
import dataclasses
import functools

import jax
import jax.numpy as jnp
from jax import lax
from jax.experimental import pallas as pl
from jax.experimental.pallas import tpu as pltpu
from jax.experimental.pallas import tpu_sc as plsc

_NC = 2
_NS = 16
_K = 80
_G = 64
_ZR = 16


def _dotT(a, b):
  return lax.dot_general(a, b, (((1,), (1,)), ((), ())),
                         preferred_element_type=jnp.float32,
                         precision=lax.Precision.HIGHEST)


def _sc_segsum(u, src, dst, with_cnt):
  N, D = u.shape
  E = src.shape[0]
  epw = E // (_NC * _NS)
  nchunks = epw // _K
  rps = (N // _NS) // 8 * 8
  rem = N - rps * _NS
  lanes = D // 16
  nsub = _K // 16
  CB = N
  AGG = (N + N // 8 + 16 * _NS * 2 - 1) // (16 * _NS) * (16 * _NS)
  zrows = AGG // _NS
  NCNT = 640 * _NS

  mesh = plsc.VectorSubcoreMesh(core_axis_name="c", subcore_axis_name="s")

  out_type = [jax.ShapeDtypeStruct((_NC, N, D), jnp.float32)]
  scratch = [
      pltpu.VMEM_SHARED((AGG if with_cnt else N, D), jnp.float32),
      pltpu.VMEM((_K,), jnp.int32),
      pltpu.VMEM((_K,), jnp.int32),
      pltpu.VMEM((_K,), jnp.int32),
      pltpu.VMEM((_K,), jnp.int32),
      pltpu.VMEM((_K, D), jnp.float32),
      pltpu.VMEM((_K, D), jnp.float32),
      pltpu.VMEM((_ZR, D), jnp.float32),
      pltpu.SemaphoreType.DMA,
      pltpu.SemaphoreType.DMA,
      pltpu.SemaphoreType.DMA,
      pltpu.SemaphoreType.DMA,
  ]
  if with_cnt:
    out_type.append(jax.ShapeDtypeStruct((_NC, NCNT, D), jnp.float32))
    scratch += [
        pltpu.VMEM((_K,), jnp.int32),
        pltpu.VMEM((_K,), jnp.int32),
        pltpu.VMEM((_K, D), jnp.float32),
        pltpu.VMEM((16, D), jnp.float32),
        pltpu.VMEM((16, D), jnp.float32),
        pltpu.SemaphoreType.DMA,
    ]

  cp = pltpu.CompilerParams()
  if "needs_layout_passes" in pltpu.CompilerParams.__dataclass_fields__:
    cp = dataclasses.replace(cp, needs_layout_passes=False)

  @functools.partial(pl.kernel, out_type=out_type, mesh=mesh,
                     scratch_types=scratch, compiler_params=cp)
  def k(*refs):
    if with_cnt:
      (u_hbm, src_hbm, dst_hbm, out_hbm, cnt_hbm,
       agg_sh, srcv0, dstv0, srcv1, dstv1, rows0, rows1, zbuf,
       si0, si1, sg0, sg1, cidxv, lane0v, cntrows, cbuf, ebuf, scnt) = refs
    else:
      (u_hbm, src_hbm, dst_hbm, out_hbm,
       agg_sh, srcv0, dstv0, srcv1, dstv1, rows0, rows1, zbuf,
       si0, si1, sg0, sg1) = refs

    c = lax.axis_index("c")
    s = lax.axis_index("s")
    ebase = (c * _NS + s) * epw
    rbase = s * rps

    zv = jnp.zeros((16,), jnp.float32)
    ov = jnp.ones((16,), jnp.float32)

    @pl.loop(0, _ZR)
    def _(r):
      @pl.loop(0, lanes)
      def _(j):
        zbuf[r, pl.ds(j * 16, 16)] = zv

    nzt = (zrows if with_cnt else rps + rem) // _ZR
    zb = s * zrows if with_cnt else rbase

    @pl.loop(0, nzt)
    def _(t):
      pltpu.async_copy(zbuf, agg_sh.at[pl.ds(zb + t * _ZR, _ZR)], sg0)

    @pl.loop(0, nzt)
    def _(t):
      pltpu.make_async_copy(zbuf, agg_sh.at[pl.ds(zb + t * _ZR, _ZR)],
                            sg0).wait()

    if with_cnt:
      @pl.loop(0, _K)
      def _(r):
        @pl.loop(0, lanes)
        def _(j):
          cntrows[r, pl.ds(j * 16, 16)] = zv

      cbv = jnp.full((16,), CB, jnp.int32)
      z16 = jnp.zeros((16,), jnp.int32)
      for j in range(nsub):
        cidxv[pl.ds(j * 16, 16)] = cbv
        lane0v[pl.ds(j * 16, 16)] = z16
      pltpu.async_copy(cntrows, agg_sh.at[cidxv], scnt, add=True)

    plsc.subcore_barrier()

    def scatter_chunk(rows_b, dstv_b):
      pltpu.sync_copy(rows_b, agg_sh.at[dstv_b], add=True)
      if with_cnt:
        pltpu.make_async_copy(cntrows, agg_sh.at[cidxv], scnt).wait()

        @pl.loop(0, nsub)
        def _(j):
          rowi = lax.broadcasted_iota(jnp.int32, (16,), 0) + j * 16
          l0 = lane0v[pl.ds(j * 16, 16)]
          for kk in range(16):
            plsc.store_scatter(cntrows, [rowi, l0 + kk], zv)
          d = dstv_b[pl.ds(j * 16, 16)]
          cidxv[pl.ds(j * 16, 16)] = lax.shift_right_logical(d, 3) + CB
          lane0 = (d & 7) * 16
          lane0v[pl.ds(j * 16, 16)] = lane0
          for kk in range(16):
            plsc.store_scatter(cntrows, [rowi, lane0 + kk], ov)

        pltpu.async_copy(cntrows, agg_sh.at[cidxv], scnt, add=True)

    def start_idx(base, sv, dv, sem):
      pltpu.async_copy(src_hbm.at[pl.ds(base, _K)], sv, sem)
      pltpu.async_copy(dst_hbm.at[pl.ds(base, _K)], dv, sem)

    def wait_idx(sv, dv, sem):
      pltpu.make_async_copy(src_hbm.at[pl.ds(0, _K)], sv, sem).wait()
      pltpu.make_async_copy(dst_hbm.at[pl.ds(0, _K)], dv, sem).wait()

    def wait_gather(rows_b, sem):
      pltpu.make_async_copy(u_hbm.at[pl.ds(0, _K)], rows_b, sem).wait()

    assert nchunks % 2 == 1
    npairs = nchunks // 2

    start_idx(ebase, srcv0, dstv0, si0)
    start_idx(ebase + _K, srcv1, dstv1, si1)
    wait_idx(srcv0, dstv0, si0)
    pltpu.async_copy(u_hbm.at[srcv0], rows0, sg0)

    @pl.loop(0, npairs)
    def _(t):
      base_c = ebase + (2 * t + 2) * _K
      base_d = ebase + (2 * t + 3) * _K
      wait_gather(rows0, sg0)
      wait_idx(srcv1, dstv1, si1)
      pltpu.async_copy(u_hbm.at[srcv1], rows1, sg1)
      scatter_chunk(rows0, dstv0)
      start_idx(base_c, srcv0, dstv0, si0)
      wait_gather(rows1, sg1)
      wait_idx(srcv0, dstv0, si0)
      pltpu.async_copy(u_hbm.at[srcv0], rows0, sg0)
      scatter_chunk(rows1, dstv1)

      @pl.when(t < npairs - 1)
      def _():
        start_idx(base_d, srcv1, dstv1, si1)

    wait_gather(rows0, sg0)
    scatter_chunk(rows0, dstv0)
    if with_cnt:
      pltpu.make_async_copy(cntrows, agg_sh.at[cidxv], scnt).wait()

    plsc.subcore_barrier()

    pltpu.sync_copy(agg_sh.at[pl.ds(rbase, rps + rem)],
                    out_hbm.at[c].at[pl.ds(rbase, rps + rem)])

    if with_cnt:
      nbase = s * 640

      @pl.loop(0, 5)
      def _(t):
        pltpu.sync_copy(agg_sh.at[pl.ds(CB + s * 80 + t * 16, 16)], cbuf)
        for g in range(8):
          for r in range(16):
            cv = cbuf[2 * g + r // 8, pl.ds((r % 8) * 16, 16)]
            for j in range(lanes):
              ebuf[r, pl.ds(j * 16, 16)] = cv
          pltpu.sync_copy(
              ebuf, cnt_hbm.at[c].at[pl.ds(nbase + t * 128 + g * 16, 16)])

  res = k(u, src, dst)
  return (res[0], res[1]) if with_cnt else res[0]


def _tc_a(x, Wr, bl, R):
  N, D = x.shape
  H = Wr.shape[0]
  nb = N // R

  def body(x_ref, wr_ref, bl_ref, r_ref):
    r_ref[...] = _dotT(x_ref[...], wr_ref[...]) + bl_ref[...]

  return pl.pallas_call(
      body,
      grid=(nb,),
      in_specs=[
          pl.BlockSpec((R, D), lambda i: (i, 0)),
          pl.BlockSpec((H, D), lambda i: (0, 0)),
          pl.BlockSpec((1, H), lambda i: (0, 0)),
      ],
      out_specs=pl.BlockSpec((R, H), lambda i: (i, 0)),
      out_shape=jax.ShapeDtypeStruct((N, H), jnp.float32),
  )(x, Wr, bl.reshape(1, H))


def _finish_layer(p_ref, cnt_ref, r_ref, wl_ref):
  p = p_ref[...]
  s = p[0] + p[1]
  cnt = cnt_ref[...]
  c2 = cnt[0] + cnt[1]
  m = s / jnp.maximum(c2, 1.0)
  return jnp.maximum(_dotT(m, wl_ref[...]) + r_ref[...], 0.0)


def _tc_b(p, cnt, r1, Wl1, Wr2, bl2, R):
  _, N, H = p.shape
  nb = N // R

  def body(p_ref, cnt_ref, r1_ref, wl_ref, wr_ref, bl_ref, h_ref, r_ref):
    h = _finish_layer(p_ref, cnt_ref, r1_ref, wl_ref)
    h_ref[...] = h
    r_ref[...] = _dotT(h, wr_ref[...]) + bl_ref[...]

  return pl.pallas_call(
      body,
      grid=(nb,),
      in_specs=[
          pl.BlockSpec((2, R, H), lambda i: (0, i, 0)),
          pl.BlockSpec((2, R, H), lambda i: (0, i, 0)),
          pl.BlockSpec((R, H), lambda i: (i, 0)),
          pl.BlockSpec((H, H), lambda i: (0, 0)),
          pl.BlockSpec((H, H), lambda i: (0, 0)),
          pl.BlockSpec((1, H), lambda i: (0, 0)),
      ],
      out_specs=[
          pl.BlockSpec((R, H), lambda i: (i, 0)),
          pl.BlockSpec((R, H), lambda i: (i, 0)),
      ],
      out_shape=[
          jax.ShapeDtypeStruct((N, H), jnp.float32),
          jax.ShapeDtypeStruct((N, H), jnp.float32),
      ],
  )(p, cnt, r1, Wl1, Wr2, bl2.reshape(1, H))


def _tc_c(p, cnt, r2, Wl2, batch3, Wfc, bfc, R):
  _, N, H = p.shape
  C = Wfc.shape[0]
  nb = N // R

  def body(p_ref, cnt_ref, r2_ref, wl_ref, b_ref, wfc_ref, bfc_ref, out_ref,
           acc, cntg):
    i = pl.program_id(0)

    @pl.when(i == 0)
    def _():
      acc[...] = jnp.zeros_like(acc)
      cntg[...] = jnp.zeros_like(cntg)

    h = _finish_layer(p_ref, cnt_ref, r2_ref, wl_ref)
    b = b_ref[0]
    gid = lax.broadcasted_iota(jnp.int32, (_G, R), 0)
    oh = (jnp.broadcast_to(b, (_G, R)) == gid).astype(jnp.float32)
    acc[...] += lax.dot_general(oh, h, (((1,), (0,)), ((), ())),
                                preferred_element_type=jnp.float32,
                                precision=lax.Precision.HIGHEST)
    cntg[...] += jnp.broadcast_to(jnp.sum(oh, axis=1, keepdims=True), (_G, H))

    @pl.when(i == nb - 1)
    def _():
      pooled = acc[...] / jnp.maximum(cntg[...], 1.0)
      out_ref[...] = _dotT(pooled, wfc_ref[...]) + bfc_ref[...]

  return pl.pallas_call(
      body,
      grid=(nb,),
      in_specs=[
          pl.BlockSpec((2, R, H), lambda i: (0, i, 0)),
          pl.BlockSpec((2, R, H), lambda i: (0, i, 0)),
          pl.BlockSpec((R, H), lambda i: (i, 0)),
          pl.BlockSpec((H, H), lambda i: (0, 0)),
          pl.BlockSpec((1, 1, R), lambda i: (i, 0, 0)),
          pl.BlockSpec((C, H), lambda i: (0, 0)),
          pl.BlockSpec((1, C), lambda i: (0, 0)),
      ],
      out_specs=pl.BlockSpec((_G, C), lambda i: (0, 0)),
      out_shape=jax.ShapeDtypeStruct((_G, C), jnp.float32),
      scratch_shapes=[
          pltpu.VMEM((_G, H), jnp.float32),
          pltpu.VMEM((_G, H), jnp.float32),
      ],
  )(p, cnt, r2, Wl2, batch3, Wfc, bfc.reshape(1, C))


def kernel(x, edge_index, batch, Wl1, bl1, Wr1, Wl2, bl2, Wr2, Wfc, bfc):
  N, D = x.shape
  R = 1000

  src = edge_index[0]
  dst = edge_index[1]
  batch3 = batch.reshape(N // R, 1, R)

  s1, cnt = _sc_segsum(x, src, dst, with_cnt=True)
  r1 = _tc_a(x, Wr1, bl1, R)
  h1, r2 = _tc_b(s1, cnt, r1, Wl1, Wr2, bl2, R)
  s2 = _sc_segsum(h1, src, dst, with_cnt=False)
  return _tc_c(s2, cnt, r2, Wl2, batch3, Wfc, bfc, R)
